# baseline (speedup 1.0000x reference)
import jax
import jax.numpy as jnp
from jax.experimental import pallas as pl
from jax.experimental.pallas import tpu as pltpu

S = 2048
D = 2048
H, DH = 16, 128
HI, DI = 4, 64
TOPK = 512
NEG = -1e30

QB = 256
RB = 256
KB = 512
PCOLS = 3 * H * DH + HI * DI + 128


def _bf(a):
    return a.astype(jnp.bfloat16)


def _proj_kernel(x_ref, wq_ref, wk_ref, wv_ref, wiq_ref, wik_ref, o_ref):
    xb = _bf(x_ref[...])
    for lo, w_ref in ((0, wq_ref), (2048, wk_ref), (4096, wv_ref),
                      (6144, wiq_ref), (6400, wik_ref)):
        o_ref[:, lo:lo + w_ref.shape[1]] = _bf(
            jnp.dot(xb, w_ref[...], preferred_element_type=jnp.float32))
    o_ref[:, 6464:] = jnp.zeros((RB, PCOLS - 6464), jnp.bfloat16)


def _attn_kernel(q_ref, k_ref, v_ref, iq_ref, ik_ref, wih_ref, wo_ref, o_ref,
                 madd_ref):
    i = pl.program_id(0)

    ik = ik_ref[:, :DI]
    isc = None
    for h in range(HI):
        iq_h = iq_ref[:, h * DI:(h + 1) * DI]
        s = jax.lax.dot_general(iq_h, ik, (((1,), (1,)), ((), ())),
                                preferred_element_type=jnp.float32)
        r = (_bf(jnp.maximum(s, 0.0)).astype(jnp.float32)
             * _bf(wih_ref[0, h]).astype(jnp.float32))
        isc = r if isc is None else isc + r

    rows = i * QB + jax.lax.broadcasted_iota(jnp.int32, (QB, S), 0)
    cols = jax.lax.broadcasted_iota(jnp.int32, (QB, S), 1)
    causal = cols <= rows
    isc = jnp.where(causal, isc, NEG)

    bits = jax.lax.bitcast_convert_type(isc, jnp.int32)
    key = jnp.where(bits < 0, bits ^ jnp.int32(0x7FFFFFFF), bits)
    cand = jnp.full((QB, 1), jnp.int32(-2**31))
    for b in range(31, -1, -1):
        mask_b = jnp.int32(-2**31) if b == 31 else jnp.int32(1 << b)
        trial = cand ^ mask_b
        cnt = jnp.sum((key >= trial).astype(jnp.int32), axis=1,
                      keepdims=True)
        cand = jnp.where(cnt >= TOPK, trial, cand)
    sel = (key >= cand) & causal
    madd_ref[...] = jnp.where(sel, 0.0, NEG).astype(jnp.float32)

    scale = 1.0 / jnp.sqrt(jnp.float32(DH))
    nch = (i * QB + QB + KB - 1) // KB
    outs = []
    for h in range(H):
        q_h = q_ref[:, h * DH:(h + 1) * DH]

        def body(j, carry):
            m_run, den, acc = carry
            k_c = k_ref[pl.ds(j * KB, KB), h * DH:(h + 1) * DH]
            v_c = v_ref[pl.ds(j * KB, KB), h * DH:(h + 1) * DH]
            lg = jax.lax.dot_general(q_h, k_c, (((1,), (1,)), ((), ())),
                                     preferred_element_type=jnp.float32)
            lg = lg * scale + madd_ref[:, pl.ds(j * KB, KB)]
            m_new = jnp.maximum(m_run, jnp.max(lg, axis=1, keepdims=True))
            alpha = jnp.exp(m_run - m_new)
            p = jnp.exp(lg - m_new)
            den = den * alpha + jnp.sum(p, axis=1, keepdims=True)
            acc = acc * alpha + jnp.dot(_bf(p), v_c,
                                        preferred_element_type=jnp.float32)
            return m_new, den, acc

        m0 = jnp.full((QB, 1), NEG, jnp.float32)
        d0 = jnp.zeros((QB, 1), jnp.float32)
        a0 = jnp.zeros((QB, DH), jnp.float32)
        m_f, den_f, acc_f = jax.lax.fori_loop(0, nch, body, (m0, d0, a0))
        outs.append(_bf(acc_f / den_f))
    ob = jnp.concatenate(outs, axis=1)
    o_ref[...] = jnp.dot(ob, wo_ref[...], preferred_element_type=jnp.float32)


def kernel(x, wq, wk, wv, wo, wiq, wik, w_ih):
    qkv = pl.pallas_call(
        _proj_kernel,
        grid=(S // RB,),
        in_specs=[
            pl.BlockSpec((RB, D), lambda j: (j, 0)),
            pl.BlockSpec((D, H * DH), lambda j: (0, 0)),
            pl.BlockSpec((D, H * DH), lambda j: (0, 0)),
            pl.BlockSpec((D, H * DH), lambda j: (0, 0)),
            pl.BlockSpec((D, HI * DI), lambda j: (0, 0)),
            pl.BlockSpec((D, DI), lambda j: (0, 0)),
        ],
        out_specs=pl.BlockSpec((RB, PCOLS), lambda j: (j, 0)),
        out_shape=jax.ShapeDtypeStruct((S, PCOLS), jnp.bfloat16),
    )(x[0], _bf(wq), _bf(wk), _bf(wv), _bf(wiq), _bf(wik))

    wih2 = jnp.pad(w_ih.reshape(1, HI), ((0, 0), (0, 128 - HI)))
    out = pl.pallas_call(
        _attn_kernel,
        grid=(S // QB,),
        in_specs=[
            pl.BlockSpec((QB, H * DH), lambda i: (i, 0)),
            pl.BlockSpec((S, H * DH), lambda i: (0, 1)),
            pl.BlockSpec((S, H * DH), lambda i: (0, 2)),
            pl.BlockSpec((QB, HI * DI), lambda i: (i, 24)),
            pl.BlockSpec((S, 128), lambda i: (0, 50)),
            pl.BlockSpec((1, 128), lambda i: (0, 0)),
            pl.BlockSpec((D, D), lambda i: (0, 0)),
        ],
        out_specs=pl.BlockSpec((QB, D), lambda i: (i, 0)),
        out_shape=jax.ShapeDtypeStruct((S, D), jnp.float32),
        scratch_shapes=[pltpu.VMEM((QB, S), jnp.float32)],
    )(qkv, qkv, qkv, qkv, qkv, wih2, _bf(wo))
    return out.reshape(1, S, D)


# speedup vs baseline: 1.1808x; 1.1808x over previous
import jax
import jax.numpy as jnp
from jax.experimental import pallas as pl
from jax.experimental.pallas import tpu as pltpu

S = 2048
D = 2048
H, DH = 16, 128
HI, DI = 4, 64
TOPK = 512
NEG = -1e30

QB = 256
RB = 256
KB = 512
PCOLS = 3 * H * DH + HI * DI + 128


def _bf(a):
    return a.astype(jnp.bfloat16)


def _proj_kernel(x_ref, wq_ref, wk_ref, wv_ref, wiq_ref, wik_ref, o_ref):
    xb = _bf(x_ref[...])
    for lo, w_ref in ((0, wq_ref), (2048, wk_ref), (4096, wv_ref),
                      (6144, wiq_ref), (6400, wik_ref)):
        o_ref[:, lo:lo + w_ref.shape[1]] = _bf(
            jnp.dot(xb, w_ref[...], preferred_element_type=jnp.float32))
    o_ref[:, 6464:] = jnp.zeros((RB, PCOLS - 6464), jnp.bfloat16)


def _attn_kernel(q_ref, k_ref, v_ref, iq_ref, ik_ref, wih_ref, wo_ref, o_ref):
    i = pl.program_id(0)

    ik = ik_ref[:, :DI]
    isc = None
    for h in range(HI):
        iq_h = iq_ref[:, h * DI:(h + 1) * DI]
        s = jax.lax.dot_general(iq_h, ik, (((1,), (1,)), ((), ())),
                                preferred_element_type=jnp.float32)
        r = (_bf(jnp.maximum(s, 0.0)).astype(jnp.float32)
             * _bf(wih_ref[0, h]).astype(jnp.float32))
        isc = r if isc is None else isc + r

    rows = i * QB + jax.lax.broadcasted_iota(jnp.int32, (QB, S), 0)
    cols = jax.lax.broadcasted_iota(jnp.int32, (QB, S), 1)
    causal = cols <= rows
    isc = jnp.where(causal, isc, NEG)

    bits = jax.lax.bitcast_convert_type(isc, jnp.int32)
    key = jnp.where(bits < 0, bits ^ jnp.int32(0x7FFFFFFF), bits)
    cand = jnp.full((QB, 1), jnp.int32(-2**31))
    for b in range(31, -1, -1):
        mask_b = jnp.int32(-2**31) if b == 31 else jnp.int32(1 << b)
        trial = cand ^ mask_b
        cnt = jnp.sum((key >= trial).astype(jnp.int32), axis=1,
                      keepdims=True)
        cand = jnp.where(cnt >= TOPK, trial, cand)
    sel = (key >= cand) & causal
    madd = jnp.where(sel, 0.0, NEG).astype(jnp.float32)

    scale = 1.0 / jnp.sqrt(jnp.float32(DH))
    outs = []
    for h in range(H):
        q_h = q_ref[:, h * DH:(h + 1) * DH]
        k_h = k_ref[:, h * DH:(h + 1) * DH]
        v_h = v_ref[:, h * DH:(h + 1) * DH]
        logits = jax.lax.dot_general(q_h, k_h, (((1,), (1,)), ((), ())),
                                     preferred_element_type=jnp.float32)
        logits = logits * scale + madd
        m = jnp.max(logits, axis=1, keepdims=True)
        p = jnp.exp(logits - m)
        denom = jnp.sum(p, axis=1, keepdims=True)
        o_h = jnp.dot(_bf(p), v_h,
                      preferred_element_type=jnp.float32) / denom
        outs.append(_bf(o_h))
    ob = jnp.concatenate(outs, axis=1)
    o_ref[...] = jnp.dot(ob, wo_ref[...], preferred_element_type=jnp.float32)


def kernel(x, wq, wk, wv, wo, wiq, wik, w_ih):
    qkv = pl.pallas_call(
        _proj_kernel,
        grid=(S // RB,),
        in_specs=[
            pl.BlockSpec((RB, D), lambda j: (j, 0)),
            pl.BlockSpec((D, H * DH), lambda j: (0, 0)),
            pl.BlockSpec((D, H * DH), lambda j: (0, 0)),
            pl.BlockSpec((D, H * DH), lambda j: (0, 0)),
            pl.BlockSpec((D, HI * DI), lambda j: (0, 0)),
            pl.BlockSpec((D, DI), lambda j: (0, 0)),
        ],
        out_specs=pl.BlockSpec((RB, PCOLS), lambda j: (j, 0)),
        out_shape=jax.ShapeDtypeStruct((S, PCOLS), jnp.bfloat16),
    )(x[0], _bf(wq), _bf(wk), _bf(wv), _bf(wiq), _bf(wik))

    wih2 = jnp.pad(w_ih.reshape(1, HI), ((0, 0), (0, 128 - HI)))
    out = pl.pallas_call(
        _attn_kernel,
        grid=(S // QB,),
        in_specs=[
            pl.BlockSpec((QB, H * DH), lambda i: (i, 0)),
            pl.BlockSpec((S, H * DH), lambda i: (0, 1)),
            pl.BlockSpec((S, H * DH), lambda i: (0, 2)),
            pl.BlockSpec((QB, HI * DI), lambda i: (i, 24)),
            pl.BlockSpec((S, 128), lambda i: (0, 50)),
            pl.BlockSpec((1, 128), lambda i: (0, 0)),
            pl.BlockSpec((D, D), lambda i: (0, 0)),
        ],
        out_specs=pl.BlockSpec((QB, D), lambda i: (i, 0)),
        out_shape=jax.ShapeDtypeStruct((S, D), jnp.float32),
    )(qkv, qkv, qkv, qkv, qkv, wih2, _bf(wo))
    return out.reshape(1, S, D)
